# Initial kernel scaffold; baseline (speedup 1.0000x reference)
import jax
import jax.numpy as jnp
from jax.experimental import pallas as pl


def kernel(x, edge_index, W_start, b_start, ln_g0, ln_b0, Wc0, bc0, alpha0, ln_g1, ln_b1, Wc1, bc1, alpha1, W_end, b_end):
    raise NotImplementedError("write your pallas kernel here")


# speedup vs baseline: 12.6548x; 12.6548x over previous
import functools

import jax
import jax.numpy as jnp
from jax import lax
from jax.experimental import pallas as pl
from jax.experimental.pallas import tpu as pltpu
from jax.experimental.pallas import tpu_sc as plsc

NC = 2
NS = 16
K = 80
RB = 1000


def _zero_slices(n):
    chunk = 8 * ((n + NS * 8 - 1) // (NS * 8))
    out = []
    for s in range(NS):
        off = s * chunk
        sz = max(0, min(chunk, n - off))
        out.append((off, sz))
    return out


def _make_deg_kernel(E, N):
    EPW = E // (NC * NS)
    assert E == EPW * NC * NS and EPW % K == 0
    slices = _zero_slices(N)
    mesh = plsc.VectorSubcoreMesh(
        core_axis_name="c", subcore_axis_name="s", num_cores=NC, num_subcores=NS
    )

    zcap = 16 * ((max(sz for _, sz in slices) + 15) // 16)
    scratch_types = [
            pltpu.VMEM((K,), jnp.int32),
            pltpu.VMEM((K,), jnp.float32),
            pltpu.VMEM((zcap,), jnp.float32),
            pltpu.VMEM_SHARED((N,), jnp.float32),
        ]

    @functools.partial(
        pl.kernel,
        out_type=[
            jax.ShapeDtypeStruct((N,), jnp.float32),
            jax.ShapeDtypeStruct((N,), jnp.float32),
        ],
        mesh=mesh,
        scratch_types=scratch_types,
    )
    def deg_kernel(dst_hbm, deg_a, deg_b, idx_v, ones_v, zbuf, deg_sh):
        c = lax.axis_index("c")
        s = lax.axis_index("s")
        for i in range(K // 16):
            ones_v[pl.ds(i * 16, 16)] = jnp.ones((16,), jnp.float32)

        def zfill(i, carry):
            zbuf[pl.ds(i * 16, 16)] = jnp.zeros((16,), jnp.float32)
            return carry

        lax.fori_loop(0, zcap // 16, zfill, 0)
        for t, (off, sz) in enumerate(slices):
            if sz > 0:
                @pl.when(s == t)
                def _():
                    pltpu.sync_copy(zbuf.at[pl.ds(0, sz)],
                                    deg_sh.at[pl.ds(off, sz)])
        plsc.subcore_barrier()
        base = (s * NC + c) * EPW

        def body(j, carry):
            pltpu.sync_copy(dst_hbm.at[pl.ds(base + j * K, K)], idx_v)
            pltpu.sync_copy(ones_v, deg_sh.at[idx_v], add=True)
            return carry

        lax.fori_loop(0, EPW // K, body, 0)
        plsc.subcore_barrier()
        for t, (off, sz) in enumerate(slices):
            if sz > 0:
                @pl.when(s == t)
                def _():
                    pltpu.sync_copy(deg_sh.at[pl.ds(off, sz)],
                                    zbuf.at[pl.ds(0, sz)])
                    @pl.when(c == 0)
                    def _():
                        pltpu.sync_copy(zbuf.at[pl.ds(0, sz)],
                                        deg_a.at[pl.ds(off, sz)])
                    @pl.when(c == 1)
                    def _():
                        pltpu.sync_copy(zbuf.at[pl.ds(0, sz)],
                                        deg_b.at[pl.ds(off, sz)])

    return deg_kernel


def _make_edge_kernel(E, N, D):
    EPW = E // (NC * NS)
    RPT = N // NS
    assert N == RPT * NS and EPW % K == 0
    mesh = plsc.VectorSubcoreMesh(
        core_axis_name="c", subcore_axis_name="s", num_cores=NC, num_subcores=NS
    )

    PR = 16
    NP = N // PR
    assert N == NP * PR

    @functools.partial(
        pl.kernel,
        out_type=[
            jax.ShapeDtypeStruct((N, D), jnp.float32),
            jax.ShapeDtypeStruct((N, D), jnp.float32),
        ],
        mesh=mesh,
        scratch_types=[
            pltpu.VMEM((K,), jnp.int32),
            pltpu.VMEM((K,), jnp.int32),
            pltpu.VMEM((K, D), jnp.float32),
            pltpu.VMEM((PR, D), jnp.float32),
            pltpu.VMEM_SHARED((N, D), jnp.float32),
            pltpu.SemaphoreType.DMA,
        ],
    )
    def edge_kernel(src_hbm, dst_hbm, g_hbm, acc_a, acc_b,
                    src_v, dst_v, rows_v, stage_v, acc_sh, sem):
        c = lax.axis_index("c")
        s = lax.axis_index("s")
        nz = PR * D // 16
        cols = D // 16

        def zfill(i, carry):
            stage_v[i // cols, pl.ds((i % cols) * 16, 16)] = (
                jnp.zeros((16,), jnp.float32))
            return carry

        lax.fori_loop(0, nz, zfill, 0)
        trips = (NP - 1 - s) // NS + 1

        def init(i, carry):
            p = s + i * NS
            pltpu.sync_copy(stage_v, acc_sh.at[pl.ds(p * PR, PR)])
            return carry

        lax.fori_loop(0, trips, init, 0)
        plsc.subcore_barrier()
        base = (s * NC + c) * EPW

        def body(j, carry):
            off = base + j * K
            pltpu.sync_copy(src_hbm.at[pl.ds(off, K)], src_v)
            pltpu.sync_copy(dst_hbm.at[pl.ds(off, K)], dst_v)
            pltpu.async_copy(g_hbm.at[src_v], rows_v, sem).wait()
            pltpu.sync_copy(rows_v, acc_sh.at[dst_v], add=True)
            return carry

        lax.fori_loop(0, EPW // K, body, 0)
        plsc.subcore_barrier()

        def writeout(i, carry):
            p = s + i * NS
            pltpu.sync_copy(acc_sh.at[pl.ds(p * PR, PR)], stage_v)
            @pl.when(c == 0)
            def _():
                pltpu.sync_copy(stage_v, acc_a.at[pl.ds(p * PR, PR)])
            @pl.when(c == 1)
            def _():
                pltpu.sync_copy(stage_v, acc_b.at[pl.ds(p * PR, PR)])
            return carry

        lax.fori_loop(0, trips, writeout, 0)

    return edge_kernel


def _silu(x):
    return x * jax.nn.sigmoid(x)


def _ln(x, g, b):
    m = jnp.mean(x, axis=1, keepdims=True)
    v = jnp.mean((x - m) ** 2, axis=1, keepdims=True)
    return (x - m) / jnp.sqrt(v + 1e-5) * g + b


def _stage_a_body(x_ref, ws_ref, bs_ref, lg_ref, lb_ref, wc_ref, dega_ref,
                  degb_ref, h0_ref, g0_ref, dinv_ref):
    h = jnp.dot(x_ref[...], ws_ref[...],
                preferred_element_type=jnp.float32) + bs_ref[...]
    h = _silu(h)
    t = jnp.dot(_ln(h, lg_ref[...], lb_ref[...]), wc_ref[...],
                preferred_element_type=jnp.float32)
    deg = dega_ref[...] + degb_ref[...] + 1.0
    dinv = lax.rsqrt(jnp.maximum(deg, 1e-12))
    h0_ref[...] = h
    g0_ref[...] = dinv * t
    dinv_ref[...] = dinv


def _stage_b_body(acca_ref, accb_ref, g0_ref, h0_ref, dinv_ref, bc_ref, a_ref,
                  lg_ref, lb_ref, wc_ref, h1_ref, g1_ref):
    dinv = dinv_ref[...]
    out0 = dinv * (acca_ref[...] + accb_ref[...] + g0_ref[...]) + bc_ref[...]
    h1 = a_ref[0, 0] * _silu(out0) + h0_ref[...]
    t = jnp.dot(_ln(h1, lg_ref[...], lb_ref[...]), wc_ref[...],
                preferred_element_type=jnp.float32)
    h1_ref[...] = h1
    g1_ref[...] = dinv * t


def _stage_c_body(acca_ref, accb_ref, g1_ref, h1_ref, dinv_ref, bc_ref, a_ref,
                  we_ref, be_ref, out_ref):
    dinv = dinv_ref[...]
    out1 = dinv * (acca_ref[...] + accb_ref[...] + g1_ref[...]) + bc_ref[...]
    h2 = a_ref[0, 0] * _silu(out1) + h1_ref[...]
    z = jnp.dot(h2, we_ref[...], preferred_element_type=jnp.float32) + be_ref[...]
    zs = z - jnp.max(z, axis=1, keepdims=True)
    out_ref[...] = zs - jnp.log(jnp.sum(jnp.exp(zs), axis=1, keepdims=True))


def _row_spec(d):
    return pl.BlockSpec((RB, d), lambda i: (i, 0))


def _full_spec(shape):
    nd = len(shape)
    return pl.BlockSpec(shape, lambda i: (0,) * nd)


def kernel(x, edge_index, W_start, b_start, ln_g0, ln_b0, Wc0, bc0, alpha0,
           ln_g1, ln_b1, Wc1, bc1, alpha1, W_end, b_end):
    N, D = x.shape
    H = W_start.shape[1]
    C = W_end.shape[1]
    E = edge_index.shape[1]
    grid = N // RB
    assert N == grid * RB

    src = edge_index[0]
    dst = edge_index[1]

    deg_a, deg_b = _make_deg_kernel(E, N)(dst)
    deg_a = deg_a.reshape(N, 1)
    deg_b = deg_b.reshape(N, 1)

    edge_kernel = _make_edge_kernel(E, N, H)

    h0, g0, dinv = pl.pallas_call(
        _stage_a_body,
        grid=(grid,),
        in_specs=[
            _row_spec(D), _full_spec((D, H)), _full_spec((1, H)),
            _full_spec((1, H)), _full_spec((1, H)), _full_spec((H, H)),
            _row_spec(1), _row_spec(1),
        ],
        out_specs=[_row_spec(H), _row_spec(H), _row_spec(1)],
        out_shape=[
            jax.ShapeDtypeStruct((N, H), jnp.float32),
            jax.ShapeDtypeStruct((N, H), jnp.float32),
            jax.ShapeDtypeStruct((N, 1), jnp.float32),
        ],
    )(x, W_start, b_start.reshape(1, H), ln_g0.reshape(1, H),
      ln_b0.reshape(1, H), Wc0, deg_a, deg_b)

    acc0a, acc0b = edge_kernel(src, dst, g0)

    h1, g1 = pl.pallas_call(
        _stage_b_body,
        grid=(grid,),
        in_specs=[
            _row_spec(H), _row_spec(H), _row_spec(H), _row_spec(H),
            _row_spec(1),
            _full_spec((1, H)), _full_spec((1, 1)),
            _full_spec((1, H)), _full_spec((1, H)), _full_spec((H, H)),
        ],
        out_specs=[_row_spec(H), _row_spec(H)],
        out_shape=[
            jax.ShapeDtypeStruct((N, H), jnp.float32),
            jax.ShapeDtypeStruct((N, H), jnp.float32),
        ],
    )(acc0a, acc0b, g0, h0, dinv, bc0.reshape(1, H), alpha0.reshape(1, 1),
      ln_g1.reshape(1, H), ln_b1.reshape(1, H), Wc1)

    acc1a, acc1b = edge_kernel(src, dst, g1)

    out = pl.pallas_call(
        _stage_c_body,
        grid=(grid,),
        in_specs=[
            _row_spec(H), _row_spec(H), _row_spec(H), _row_spec(H),
            _row_spec(1),
            _full_spec((1, H)), _full_spec((1, 1)),
            _full_spec((H, C)), _full_spec((1, C)),
        ],
        out_specs=[_row_spec(C)],
        out_shape=[jax.ShapeDtypeStruct((N, C), jnp.float32)],
    )(acc1a, acc1b, g1, h1, dinv, bc1.reshape(1, H), alpha1.reshape(1, 1),
      W_end, b_end.reshape(1, C))

    return out[0]


# speedup vs baseline: 28.5811x; 2.2585x over previous
import functools

import jax
import jax.numpy as jnp
from jax import lax
from jax.experimental import pallas as pl
from jax.experimental.pallas import tpu as pltpu
from jax.experimental.pallas import tpu_sc as plsc

NC = 2
NS = 16
K = 80
RB = 1000


def _zero_slices(n):
    chunk = 8 * ((n + NS * 8 - 1) // (NS * 8))
    out = []
    for s in range(NS):
        off = s * chunk
        sz = max(0, min(chunk, n - off))
        out.append((off, sz))
    return out


def _make_deg_kernel(E, N):
    EPW = E // (NC * NS)
    assert E == EPW * NC * NS and EPW % K == 0
    slices = _zero_slices(N)
    mesh = plsc.VectorSubcoreMesh(
        core_axis_name="c", subcore_axis_name="s", num_cores=NC, num_subcores=NS
    )

    zcap = 16 * ((max(sz for _, sz in slices) + 15) // 16)
    NCH = EPW // K
    RING = 5
    assert NCH % RING == 0
    scratch_types = (
        [pltpu.VMEM((EPW,), jnp.int32),
         pltpu.VMEM((K,), jnp.float32),
         pltpu.VMEM((zcap,), jnp.float32)]
        + [pltpu.VMEM((K,), jnp.int32) for _ in range(RING)]
        + [pltpu.VMEM_SHARED((N,), jnp.float32)]
        + [pltpu.SemaphoreType.DMA for _ in range(RING)]
    )

    @functools.partial(
        pl.kernel,
        out_type=[
            jax.ShapeDtypeStruct((N,), jnp.float32),
            jax.ShapeDtypeStruct((N,), jnp.float32),
        ],
        mesh=mesh,
        scratch_types=scratch_types,
    )
    def deg_kernel(dst_hbm, deg_a, deg_b, dst1d, ones_v, zbuf,
                   i0, i1, i2, i3, i4, deg_sh, s0, s1, s2, s3, s4):
        idxb = (i0, i1, i2, i3, i4)
        sems = (s0, s1, s2, s3, s4)
        c = lax.axis_index("c")
        s = lax.axis_index("s")
        for i in range(K // 16):
            ones_v[pl.ds(i * 16, 16)] = jnp.ones((16,), jnp.float32)

        def zfill(i, carry):
            zbuf[pl.ds(i * 16, 16)] = jnp.zeros((16,), jnp.float32)
            return carry

        lax.fori_loop(0, zcap // 16, zfill, 0)
        for t, (off, sz) in enumerate(slices):
            if sz > 0:
                @pl.when(s == t)
                def _():
                    pltpu.sync_copy(zbuf.at[pl.ds(0, sz)],
                                    deg_sh.at[pl.ds(off, sz)])
        plsc.subcore_barrier()
        base = (s * NC + c) * EPW
        pltpu.sync_copy(dst_hbm.at[pl.ds(base, EPW)], dst1d)

        def outer(i, carry):
            for b in range(RING):
                j = i * RING + b
                for k2 in range(K // 16):
                    idxb[b][pl.ds(k2 * 16, 16)] = (
                        dst1d[pl.ds(j * K + k2 * 16, 16)])
                pltpu.async_copy(ones_v, deg_sh.at[idxb[b]], sems[b],
                                 add=True)
            for b in range(RING):
                pltpu.make_async_copy(ones_v, deg_sh.at[idxb[b]],
                                      sems[b]).wait()
            return carry

        lax.fori_loop(0, NCH // RING, outer, 0)
        plsc.subcore_barrier()
        for t, (off, sz) in enumerate(slices):
            if sz > 0:
                @pl.when(s == t)
                def _():
                    pltpu.sync_copy(deg_sh.at[pl.ds(off, sz)],
                                    zbuf.at[pl.ds(0, sz)])
                    @pl.when(c == 0)
                    def _():
                        pltpu.sync_copy(zbuf.at[pl.ds(0, sz)],
                                        deg_a.at[pl.ds(off, sz)])
                    @pl.when(c == 1)
                    def _():
                        pltpu.sync_copy(zbuf.at[pl.ds(0, sz)],
                                        deg_b.at[pl.ds(off, sz)])

    return deg_kernel


def _make_edge_kernel(E, N, D):
    EPW = E // (NC * NS)
    RPT = N // NS
    assert N == RPT * NS and EPW % K == 0
    mesh = plsc.VectorSubcoreMesh(
        core_axis_name="c", subcore_axis_name="s", num_cores=NC, num_subcores=NS
    )

    PR = 16
    NP = N // PR
    NCH = EPW // K
    assert N == NP * PR and NCH % 2 == 1

    @functools.partial(
        pl.kernel,
        out_type=[
            jax.ShapeDtypeStruct((N, D), jnp.float32),
            jax.ShapeDtypeStruct((N, D), jnp.float32),
        ],
        mesh=mesh,
        scratch_types=[
            pltpu.VMEM((EPW,), jnp.int32),
            pltpu.VMEM((EPW,), jnp.int32),
            pltpu.VMEM((K,), jnp.int32),
            pltpu.VMEM((K, D), jnp.float32),
            pltpu.VMEM((K, D), jnp.float32),
            pltpu.VMEM((PR, D), jnp.float32),
            pltpu.VMEM_SHARED((N, D), jnp.float32),
            pltpu.SemaphoreType.DMA,
            pltpu.SemaphoreType.DMA,
        ],
    )
    def edge_kernel(src_hbm, dst_hbm, g_hbm, acc_a, acc_b,
                    src1d, dst1d, dst80, rows_a, rows_b, stage_v, acc_sh,
                    sem_a, sem_b):
        c = lax.axis_index("c")
        s = lax.axis_index("s")
        nz = PR * D // 16
        cols = D // 16

        def zfill(i, carry):
            stage_v[i // cols, pl.ds((i % cols) * 16, 16)] = (
                jnp.zeros((16,), jnp.float32))
            return carry

        lax.fori_loop(0, nz, zfill, 0)
        trips = (NP - 1 - s) // NS + 1

        def init(i, carry):
            p = s + i * NS
            pltpu.sync_copy(stage_v, acc_sh.at[pl.ds(p * PR, PR)])
            return carry

        lax.fori_loop(0, trips, init, 0)
        plsc.subcore_barrier()
        base = (s * NC + c) * EPW
        pltpu.sync_copy(src_hbm.at[pl.ds(base, EPW)], src1d)
        pltpu.sync_copy(dst_hbm.at[pl.ds(base, EPW)], dst1d)

        def fire(j, rows, sem):
            pltpu.async_copy(g_hbm.at[src1d.at[pl.ds(j * K, K)]], rows, sem)

        def drain(j, rows, sem):
            pltpu.make_async_copy(
                g_hbm.at[src1d.at[pl.ds(j * K, K)]], rows, sem).wait()

        def scatter(j, rows):
            for k2 in range(K // 16):
                dst80[pl.ds(k2 * 16, 16)] = dst1d[pl.ds(j * K + k2 * 16, 16)]
            pltpu.sync_copy(rows, acc_sh.at[dst80], add=True)

        fire(0, rows_a, sem_a)

        def pair(i, carry):
            j0 = 2 * i
            fire(j0 + 1, rows_b, sem_b)
            drain(j0, rows_a, sem_a)
            scatter(j0, rows_a)
            fire(j0 + 2, rows_a, sem_a)
            drain(j0 + 1, rows_b, sem_b)
            scatter(j0 + 1, rows_b)
            return carry

        lax.fori_loop(0, (NCH - 1) // 2, pair, 0)
        drain(NCH - 1, rows_a, sem_a)
        scatter(NCH - 1, rows_a)
        plsc.subcore_barrier()

        def writeout(i, carry):
            p = s + i * NS
            pltpu.sync_copy(acc_sh.at[pl.ds(p * PR, PR)], stage_v)
            @pl.when(c == 0)
            def _():
                pltpu.sync_copy(stage_v, acc_a.at[pl.ds(p * PR, PR)])
            @pl.when(c == 1)
            def _():
                pltpu.sync_copy(stage_v, acc_b.at[pl.ds(p * PR, PR)])
            return carry

        lax.fori_loop(0, trips, writeout, 0)

    return edge_kernel


def _silu(x):
    return x * jax.nn.sigmoid(x)


def _ln(x, g, b):
    m = jnp.mean(x, axis=1, keepdims=True)
    v = jnp.mean((x - m) ** 2, axis=1, keepdims=True)
    return (x - m) / jnp.sqrt(v + 1e-5) * g + b


def _stage_a_body(x_ref, ws_ref, bs_ref, lg_ref, lb_ref, wc_ref, dega_ref,
                  degb_ref, h0_ref, g0_ref, dinv_ref):
    h = jnp.dot(x_ref[...], ws_ref[...],
                preferred_element_type=jnp.float32) + bs_ref[...]
    h = _silu(h)
    t = jnp.dot(_ln(h, lg_ref[...], lb_ref[...]), wc_ref[...],
                preferred_element_type=jnp.float32)
    deg = dega_ref[...] + degb_ref[...] + 1.0
    dinv = lax.rsqrt(jnp.maximum(deg, 1e-12))
    h0_ref[...] = h
    g0_ref[...] = dinv * t
    dinv_ref[...] = dinv


def _stage_b_body(acca_ref, accb_ref, g0_ref, h0_ref, dinv_ref, bc_ref, a_ref,
                  lg_ref, lb_ref, wc_ref, h1_ref, g1_ref):
    dinv = dinv_ref[...]
    out0 = dinv * (acca_ref[...] + accb_ref[...] + g0_ref[...]) + bc_ref[...]
    h1 = a_ref[0, 0] * _silu(out0) + h0_ref[...]
    t = jnp.dot(_ln(h1, lg_ref[...], lb_ref[...]), wc_ref[...],
                preferred_element_type=jnp.float32)
    h1_ref[...] = h1
    g1_ref[...] = dinv * t


def _stage_c_body(acca_ref, accb_ref, g1_ref, h1_ref, dinv_ref, bc_ref, a_ref,
                  we_ref, be_ref, out_ref):
    dinv = dinv_ref[...]
    out1 = dinv * (acca_ref[...] + accb_ref[...] + g1_ref[...]) + bc_ref[...]
    h2 = a_ref[0, 0] * _silu(out1) + h1_ref[...]
    z = jnp.dot(h2, we_ref[...], preferred_element_type=jnp.float32) + be_ref[...]
    zs = z - jnp.max(z, axis=1, keepdims=True)
    out_ref[...] = zs - jnp.log(jnp.sum(jnp.exp(zs), axis=1, keepdims=True))


def _row_spec(d):
    return pl.BlockSpec((RB, d), lambda i: (i, 0))


def _full_spec(shape):
    nd = len(shape)
    return pl.BlockSpec(shape, lambda i: (0,) * nd)


def kernel(x, edge_index, W_start, b_start, ln_g0, ln_b0, Wc0, bc0, alpha0,
           ln_g1, ln_b1, Wc1, bc1, alpha1, W_end, b_end):
    N, D = x.shape
    H = W_start.shape[1]
    C = W_end.shape[1]
    E = edge_index.shape[1]
    grid = N // RB
    assert N == grid * RB

    src = edge_index[0]
    dst = edge_index[1]

    deg_a, deg_b = _make_deg_kernel(E, N)(dst)
    deg_a = deg_a.reshape(N, 1)
    deg_b = deg_b.reshape(N, 1)

    edge_kernel = _make_edge_kernel(E, N, H)

    h0, g0, dinv = pl.pallas_call(
        _stage_a_body,
        grid=(grid,),
        in_specs=[
            _row_spec(D), _full_spec((D, H)), _full_spec((1, H)),
            _full_spec((1, H)), _full_spec((1, H)), _full_spec((H, H)),
            _row_spec(1), _row_spec(1),
        ],
        out_specs=[_row_spec(H), _row_spec(H), _row_spec(1)],
        out_shape=[
            jax.ShapeDtypeStruct((N, H), jnp.float32),
            jax.ShapeDtypeStruct((N, H), jnp.float32),
            jax.ShapeDtypeStruct((N, 1), jnp.float32),
        ],
    )(x, W_start, b_start.reshape(1, H), ln_g0.reshape(1, H),
      ln_b0.reshape(1, H), Wc0, deg_a, deg_b)

    acc0a, acc0b = edge_kernel(src, dst, g0)

    h1, g1 = pl.pallas_call(
        _stage_b_body,
        grid=(grid,),
        in_specs=[
            _row_spec(H), _row_spec(H), _row_spec(H), _row_spec(H),
            _row_spec(1),
            _full_spec((1, H)), _full_spec((1, 1)),
            _full_spec((1, H)), _full_spec((1, H)), _full_spec((H, H)),
        ],
        out_specs=[_row_spec(H), _row_spec(H)],
        out_shape=[
            jax.ShapeDtypeStruct((N, H), jnp.float32),
            jax.ShapeDtypeStruct((N, H), jnp.float32),
        ],
    )(acc0a, acc0b, g0, h0, dinv, bc0.reshape(1, H), alpha0.reshape(1, 1),
      ln_g1.reshape(1, H), ln_b1.reshape(1, H), Wc1)

    acc1a, acc1b = edge_kernel(src, dst, g1)

    out = pl.pallas_call(
        _stage_c_body,
        grid=(grid,),
        in_specs=[
            _row_spec(H), _row_spec(H), _row_spec(H), _row_spec(H),
            _row_spec(1),
            _full_spec((1, H)), _full_spec((1, 1)),
            _full_spec((H, C)), _full_spec((1, C)),
        ],
        out_specs=[_row_spec(C)],
        out_shape=[jax.ShapeDtypeStruct((N, C), jnp.float32)],
    )(acc1a, acc1b, g1, h1, dinv, bc1.reshape(1, H), alpha1.reshape(1, 1),
      W_end, b_end.reshape(1, C))

    return out[0]
